# baseline (speedup 1.0000x reference)
import jax
import jax.numpy as jnp
from jax.experimental import pallas as pl
from jax.experimental.pallas import tpu as pltpu
from jax.experimental.pallas import tpu_sc as plsc

B, L, E, R = 1024, 200, 128, 128
EPS = 1e-7
BBA = 128
BBB = 128

_NC = 1
_NT = _NC * 16
_RPT = B // _NT
_FLAT = _RPT * L


def _matvec_kernel(lba_ref, embs_ref, w_ref, s1_ref, s2_ref):
    wsum = jnp.sum(w_ref[...], axis=1)
    w1 = jnp.broadcast_to(wsum[:R].reshape(1, 1, R), (BBA, 1, R))
    w2 = jnp.broadcast_to(wsum[R:].reshape(1, 1, E), (BBA, 1, E))
    s1 = jax.lax.dot_general(
        w1, lba_ref[...], (((2,), (2,)), ((0,), (0,))),
        preferred_element_type=jnp.float32)
    s2 = jax.lax.dot_general(
        w2, embs_ref[...], (((2,), (2,)), ((0,), (0,))),
        preferred_element_type=jnp.float32)
    s1_ref[...] = s1[:, 0, :]
    s2_ref[...] = s2[:, 0, :]


def _sc_gather_body(s1_hbm, p_hbm, g_hbm, s1_v, p_v, g_v):
    wid = jax.lax.axis_index("s") * _NC + jax.lax.axis_index("c")
    base = wid * _FLAT
    pltpu.sync_copy(s1_hbm.at[pl.ds(base, _FLAT)], s1_v)
    pltpu.sync_copy(p_hbm.at[pl.ds(base, _FLAT)], p_v)

    def body(k, carry):
        off = k * 16
        idx = p_v[pl.ds(off, 16)]
        g_v[pl.ds(off, 16)] = plsc.load_gather(s1_v, [idx])
        return carry

    jax.lax.fori_loop(0, _FLAT // 16, body, 0)
    pltpu.sync_copy(g_v, g_hbm.at[pl.ds(base, _FLAT)])


_sc_gather = pl.kernel(
    _sc_gather_body,
    out_type=jax.ShapeDtypeStruct((B * L,), jnp.float32),
    mesh=plsc.VectorSubcoreMesh(
        core_axis_name="c", subcore_axis_name="s", num_cores=_NC),
    compiler_params=pltpu.CompilerParams(needs_layout_passes=False),
    scratch_types=[
        pltpu.VMEM((_FLAT,), jnp.float32),
        pltpu.VMEM((_FLAT,), jnp.int32),
        pltpu.VMEM((_FLAT,), jnp.float32),
    ],
)


def _out_kernel(g_ref, s2_ref, rnn_ref, out_ref):
    a = jnp.exp(jnp.tanh(g_ref[...] + s2_ref[...]))
    num = jax.lax.dot_general(
        a[:, None, :], rnn_ref[...], (((2,), (1,)), ((0,), (0,))),
        preferred_element_type=jnp.float32)
    denom = jnp.sum(a, axis=1)[:, None] + EPS
    out_ref[...] = num[:, 0, :] / denom


def kernel(embs, prnt_indices, lba_out, rnn_out, W):
    s1, s2 = pl.pallas_call(
        _matvec_kernel,
        grid=(B // BBA,),
        in_specs=[
            pl.BlockSpec((BBA, L, R), lambda i: (i, 0, 0)),
            pl.BlockSpec((BBA, L, E), lambda i: (i, 0, 0)),
            pl.BlockSpec((E + R, R), lambda i: (0, 0)),
        ],
        out_specs=[
            pl.BlockSpec((BBA, L), lambda i: (i, 0)),
            pl.BlockSpec((BBA, L), lambda i: (i, 0)),
        ],
        out_shape=[
            jax.ShapeDtypeStruct((B, L), jnp.float32),
            jax.ShapeDtypeStruct((B, L), jnp.float32),
        ],
    )(lba_out, embs, W)

    lp = (jnp.arange(B, dtype=jnp.int32)[:, None] % _RPT) * L + prnt_indices
    g = _sc_gather(s1.reshape(B * L), lp.reshape(B * L))

    return pl.pallas_call(
        _out_kernel,
        grid=(B // BBB,),
        in_specs=[
            pl.BlockSpec((BBB, L), lambda i: (i, 0)),
            pl.BlockSpec((BBB, L), lambda i: (i, 0)),
            pl.BlockSpec((BBB, L, R), lambda i: (i, 0, 0)),
        ],
        out_specs=pl.BlockSpec((BBB, R), lambda i: (i, 0)),
        out_shape=jax.ShapeDtypeStruct((B, R), jnp.float32),
    )(g.reshape(B, L), s2, rnn_out)


# speedup vs baseline: 1.0151x; 1.0151x over previous
import jax
import jax.numpy as jnp
from jax.experimental import pallas as pl
from jax.experimental.pallas import tpu as pltpu
from jax.experimental.pallas import tpu_sc as plsc

B, L, E, R = 1024, 200, 128, 128
EPS = 1e-7
BBA = 128
BBB = 128

_NC = 2
_NT = _NC * 16
_RPT = B // _NT
_FLAT = _RPT * L


def _matvec_kernel(lba_ref, embs_ref, w_ref, s1_ref, s2_ref):
    wsum = jnp.sum(w_ref[...], axis=1)
    w1 = jnp.broadcast_to(wsum[:R].reshape(1, 1, R), (BBA, 1, R))
    w2 = jnp.broadcast_to(wsum[R:].reshape(1, 1, E), (BBA, 1, E))
    s1 = jax.lax.dot_general(
        w1, lba_ref[...], (((2,), (2,)), ((0,), (0,))),
        preferred_element_type=jnp.float32)
    s2 = jax.lax.dot_general(
        w2, embs_ref[...], (((2,), (2,)), ((0,), (0,))),
        preferred_element_type=jnp.float32)
    s1_ref[...] = s1[:, 0, :]
    s2_ref[...] = s2[:, 0, :]


def _sc_gather_body(s1_hbm, p_hbm, g_hbm, s1_v, p_v, g_v):
    wid = jax.lax.axis_index("s") * _NC + jax.lax.axis_index("c")
    base = wid * _FLAT
    pltpu.sync_copy(s1_hbm.at[pl.ds(base, _FLAT)], s1_v)
    pltpu.sync_copy(p_hbm.at[pl.ds(base, _FLAT)], p_v)

    def body(k, carry):
        off = k * 16
        idx = p_v[pl.ds(off, 16)]
        g_v[pl.ds(off, 16)] = plsc.load_gather(s1_v, [idx])
        return carry

    jax.lax.fori_loop(0, _FLAT // 16, body, 0)
    pltpu.sync_copy(g_v, g_hbm.at[pl.ds(base, _FLAT)])


_sc_gather = pl.kernel(
    _sc_gather_body,
    out_type=jax.ShapeDtypeStruct((B * L,), jnp.float32),
    mesh=plsc.VectorSubcoreMesh(
        core_axis_name="c", subcore_axis_name="s", num_cores=_NC),
    compiler_params=pltpu.CompilerParams(needs_layout_passes=False),
    scratch_types=[
        pltpu.VMEM((_FLAT,), jnp.float32),
        pltpu.VMEM((_FLAT,), jnp.int32),
        pltpu.VMEM((_FLAT,), jnp.float32),
    ],
)


def _out_kernel(g_ref, s2_ref, rnn_ref, out_ref):
    a = jnp.exp(jnp.tanh(g_ref[...] + s2_ref[...]))
    num = jax.lax.dot_general(
        a[:, None, :], rnn_ref[...], (((2,), (1,)), ((0,), (0,))),
        preferred_element_type=jnp.float32)
    denom = jnp.sum(a, axis=1)[:, None] + EPS
    out_ref[...] = num[:, 0, :] / denom


def kernel(embs, prnt_indices, lba_out, rnn_out, W):
    s1, s2 = pl.pallas_call(
        _matvec_kernel,
        grid=(B // BBA,),
        in_specs=[
            pl.BlockSpec((BBA, L, R), lambda i: (i, 0, 0)),
            pl.BlockSpec((BBA, L, E), lambda i: (i, 0, 0)),
            pl.BlockSpec((E + R, R), lambda i: (0, 0)),
        ],
        out_specs=[
            pl.BlockSpec((BBA, L), lambda i: (i, 0)),
            pl.BlockSpec((BBA, L), lambda i: (i, 0)),
        ],
        out_shape=[
            jax.ShapeDtypeStruct((B, L), jnp.float32),
            jax.ShapeDtypeStruct((B, L), jnp.float32),
        ],
    )(lba_out, embs, W)

    lp = (jnp.arange(B, dtype=jnp.int32)[:, None] % _RPT) * L + prnt_indices
    g = _sc_gather(s1.reshape(B * L), lp.reshape(B * L))

    return pl.pallas_call(
        _out_kernel,
        grid=(B // BBB,),
        in_specs=[
            pl.BlockSpec((BBB, L), lambda i: (i, 0)),
            pl.BlockSpec((BBB, L), lambda i: (i, 0)),
            pl.BlockSpec((BBB, L, R), lambda i: (i, 0, 0)),
        ],
        out_specs=pl.BlockSpec((BBB, R), lambda i: (i, 0)),
        out_shape=jax.ShapeDtypeStruct((B, R), jnp.float32),
    )(g.reshape(B, L), s2, rnn_out)
